# baseline (speedup 1.0000x reference)
import functools

import jax
import jax.numpy as jnp
from jax import lax
from jax.experimental import pallas as pl
from jax.experimental.pallas import tpu as pltpu
from jax.experimental.pallas import tpu_sc as plsc

INF = 1e8
LEVEL_SIZES = (32768, 16384, 8192)
LEVEL_OFFS = (0, 32768, 49152)
STRIDES = (1.0, 2.0, 4.0)
B = 2
M = 256
N_TOTAL = 57344
AUDIO_TARGET_RATE = 22050.0 / 256.0
BEAT_RADIUS = 2.5
DOWNBEAT_RADIUS = 4.5
EPS = 1e-6

_EDGE0 = 0.35 + (0.7 - 0.35) / 2.0
_EDGE1 = 0.7 + (1.4 - 0.7) / 2.0
RANGE_LO = (-1.0 * AUDIO_TARGET_RATE, _EDGE0 * AUDIO_TARGET_RATE, _EDGE1 * AUDIO_TARGET_RATE)
RANGE_HI = (_EDGE0 * AUDIO_TARGET_RATE, _EDGE1 * AUDIO_TARGET_RATE, 1000.0 * AUDIO_TARGET_RATE)

NW = 32
CMAX = LEVEL_SIZES[0] // NW
_PAD = 8


def _sc_assign_body(ann_hbm, aux_hbm, out_hbm, ann_v, aux_v, ba_v, bl_v, br_v,
                    bc_v, st_v, sem):
    wid = lax.axis_index("s") * 2 + lax.axis_index("c")
    pltpu.sync_copy(ann_hbm, ann_v)
    pltpu.sync_copy(aux_hbm, aux_v)
    lane = lax.broadcasted_iota(jnp.int32, (16,), 0)
    lanef = lane.astype(jnp.float32)
    big_g = jnp.full((16,), float(M), jnp.float32)
    neg_g = jnp.full((16,), -1.0, jnp.float32)
    out_dmas = []

    for b in range(B):
        for lvl in range(3):
            C = LEVEL_SIZES[lvl] // NW
            s = STRIDES[lvl]
            inv_s = 1.0 / s
            lo = RANGE_LO[lvl]
            hi = RANGE_HI[lvl]
            base = wid * C

            def init_body(i, _):
                ba_v[pl.ds(i * 16, 16)] = jnp.full((16,), INF, jnp.float32)
                return 0

            lax.fori_loop(0, (C + 32) // 16, init_body, 0, unroll=4)

            def ann_group_body(g, _, b=b, base=base, C=C, s=s, inv_s=inv_s,
                               lo=lo, hi=hi):
                gs = pl.ds(g * 16, 16)
                lvec = ann_v[b, 0, gs]
                k0v = (lvec * inv_s).astype(jnp.int32)
                rvec = ann_v[b, 1, gs]
                cvec = ann_v[b, 2, gs]
                for j in range(16):
                    k0 = k0v[j]

                    @pl.when((k0 + 5 >= base) & (k0 < base + C))
                    def _process(j=j, k0=k0):
                        l = lvec[j]
                        r = rvec[j]
                        c = cvec[j]
                        kvec = k0 + lane
                        af = kvec.astype(jnp.float32) * s
                        zero_v = af * 0.0
                        l_v = l + zero_v
                        r_v = r + zero_v
                        c_v = c + zero_v
                        rad_s = (DOWNBEAT_RADIUS
                                 - (DOWNBEAT_RADIUS - BEAT_RADIUS) * c) * s
                        in_box = (af >= l_v) & (af <= jnp.minimum(r_v, l + rad_s + zero_v))
                        l_star = af - l_v
                        r_star = r_v - af
                        mx = jnp.maximum(l_star, r_star)
                        valid = (in_box & (mx >= lo) & (mx <= hi)
                                 & (kvec >= base) & (kvec < base + C)
                                 & (lane < 6))
                        area_v = r_v - l_v
                        off = k0 - base + _PAD
                        sl = pl.ds(off, 16)
                        ba = ba_v[sl]
                        better = valid & (area_v < ba)
                        ba_v[sl] = jnp.where(better, area_v, ba)
                        bl_v[sl] = jnp.where(better, l_v, bl_v[sl])
                        br_v[sl] = jnp.where(better, r_v, br_v[sl])
                        bc_v[sl] = jnp.where(better, c_v, bc_v[sl])

                return 0

            hf = aux_v[b, 0] * inv_s
            tf = aux_v[b, 1] * inv_s
            basef = wid * float(C)
            lim_lo = basef - 5.0
            lim_hi = basef + float(C)
            g_lo = wid * 0 + M // 16
            g_hi = wid * 0 - 1
            for g in range(M // 16 - 1, -1, -1):
                g_lo = jnp.where((tf[g] >= lim_lo) & (hf[g] < lim_hi), g, g_lo)
            for g in range(M // 16):
                g_hi = jnp.where((tf[g] >= lim_lo) & (hf[g] < lim_hi), g, g_hi)
            lax.fori_loop(g_lo, g_hi + 1, ann_group_body, 0, unroll=False)

            def epi_body(i, _, b=b, lvl=lvl, base=base, s=s, inv_s=inv_s):
                sl = pl.ds(_PAD + i * 16, 16)
                kvec = base + i * 16 + lane
                af = kvec.astype(jnp.float32) * s
                ba = ba_v[sl]
                pos = ba < INF
                posf = jnp.sign(INF - ba)
                zero_v = posf * 0.0
                bl = jnp.where(pos, bl_v[sl], zero_v)
                br = jnp.where(pos, br_v[sl], zero_v)
                bc = jnp.where(pos, bc_v[sl], zero_v)
                sbase = (b * 3 + lvl) * 3 * CMAX
                st_v[pl.ds(sbase + i * 16, 16)] = posf + bc
                st_v[pl.ds(sbase + CMAX + i * 16, 16)] = (af - bl) * inv_s
                st_v[pl.ds(sbase + 2 * CMAX + i * 16, 16)] = (br - af) * inv_s
                return 0

            lax.fori_loop(0, C // 16, epi_body, 0, unroll=4)

            gstart = LEVEL_OFFS[lvl] + base
            for f in range(3):
                lin = (b * 3 + f) * N_TOTAL + gstart
                out_dmas.append(pltpu.async_copy(
                    st_v.at[pl.ds(((b * 3 + lvl) * 3 + f) * CMAX, C)],
                    out_hbm.at[pl.ds(lin, C)], sem))

    for dma in out_dmas:
        dma.wait()


def _make_sc_assign():
    mesh = plsc.VectorSubcoreMesh(core_axis_name="c", subcore_axis_name="s")
    return pl.kernel(
        _sc_assign_body,
        out_type=jax.ShapeDtypeStruct((B * 3 * N_TOTAL,), jnp.float32),
        mesh=mesh,
        scratch_types=[
            pltpu.VMEM((B, 3, M), jnp.float32),
            pltpu.VMEM((B, 2, 16), jnp.float32),
            pltpu.VMEM((CMAX + 32,), jnp.float32),
            pltpu.VMEM((CMAX + 32,), jnp.float32),
            pltpu.VMEM((CMAX + 32,), jnp.float32),
            pltpu.VMEM((CMAX + 32,), jnp.float32),
            pltpu.VMEM((B * 3 * 3 * CMAX,), jnp.float32),
            pltpu.SemaphoreType.DMA,
        ],
    )


BLK = 14336
NBLK = N_TOTAL // BLK


def _loss_kernel(cls_ref, reg_ref, lef_ref, asg_ref, out_ref, acc_ref):
    j = pl.program_id(1)

    pc = asg_ref[0, 0]
    nl = asg_ref[0, 1]
    nr = asg_ref[0, 2]

    posf = jnp.minimum(pc, 1.0)
    t1 = jnp.maximum(pc - 1.0, 0.0)
    t0 = posf - t1
    p = jnp.clip(cls_ref[0], EPS, 1.0 - EPS)
    p0 = p[0]
    p1 = p[1]

    def _focal(t, q):
        is_pos = t == 1.0
        arg = jnp.where(is_pos, q, 1.0 - q)
        coef = jnp.where(is_pos, 0.25 * (1.0 - q) * (1.0 - q), 0.75 * q * q)
        return coef * (-jnp.log(arg))

    cls_sum = jnp.sum(_focal(t0, p0) + _focal(t1, p1))

    tl = jnp.maximum(nl, 1e-3)
    tr = jnp.maximum(nr, 1e-3)
    reg = reg_ref[0]
    pl_ = reg[0]
    pr_ = reg[1]
    inter = jnp.minimum(pl_, tl) + jnp.minimum(pr_, tr)
    union = jnp.maximum(pl_, tl) + jnp.maximum(pr_, tr)
    iou = jnp.clip(inter / (union + EPS), EPS, 1.0)
    reg_sum = jnp.sum(-jnp.log(iou) * posf)

    lt = jnp.clip(tr / (tl + tr + EPS), EPS, 1.0 - EPS)
    lp = jnp.clip(lef_ref[0, 0], EPS, 1.0 - EPS)
    lbce = -(lt * jnp.log(lp) + (1.0 - lt) * jnp.log(1.0 - lp))
    left_sum = jnp.sum(lbce * posf)

    pos_sum = jnp.sum(posf)

    b = pl.program_id(0)
    lane = lax.broadcasted_iota(jnp.int32, (1, 1, 128), 2)
    acc = jnp.where(lane == 0, cls_sum + reg_sum + left_sum,
                    jnp.where(lane == 1, pos_sum, 0.0))

    @pl.when(j == 0)
    def _init():
        acc_ref[pl.ds(b, 1)] = acc

    @pl.when(j > 0)
    def _acc():
        acc_ref[pl.ds(b, 1)] += acc

    @pl.when((b == B - 1) & (j == NBLK - 1))
    def _finalize():
        total = 0.0
        for bb in range(B):
            row = acc_ref[bb]
            lane2 = lax.broadcasted_iota(jnp.int32, (1, 128), 1)
            lsum = jnp.sum(jnp.where(lane2 == 0, row, 0.0))
            npos = jnp.maximum(jnp.sum(jnp.where(lane2 == 1, row, 0.0)), 1.0)
            total = total + lsum / npos
        out_ref[...] = jnp.where(lane[0] == 0, total / float(B), 0.0)


def kernel(classifications, regressions, leftnesses, annotations):
    cls_t = classifications.transpose(0, 2, 1)
    reg_t = regressions.transpose(0, 2, 1)
    lef_t = leftnesses.transpose(0, 2, 1)
    ann_t = annotations.transpose(0, 2, 1)
    starts_t = ann_t[:, 0, :]
    aux = jnp.stack([starts_t[:, 0::16], starts_t[:, 15::16]], axis=1)

    assign = _make_sc_assign()(ann_t, aux).reshape(B, 3, N_TOTAL)

    out = pl.pallas_call(
        _loss_kernel,
        grid=(B, NBLK),
        in_specs=[
            pl.BlockSpec((1, 2, BLK), lambda b, j: (b, 0, j)),
            pl.BlockSpec((1, 2, BLK), lambda b, j: (b, 0, j)),
            pl.BlockSpec((1, 1, BLK), lambda b, j: (b, 0, j)),
            pl.BlockSpec((1, 3, BLK), lambda b, j: (b, 0, j)),
        ],
        out_specs=pl.BlockSpec((1, 128), lambda b, j: (0, 0)),
        out_shape=jax.ShapeDtypeStruct((1, 128), jnp.float32),
        scratch_shapes=[pltpu.VMEM((B, 1, 128), jnp.float32)],
    )(cls_t, reg_t, lef_t, assign)

    return out[0, 0]


# speedup vs baseline: 1.0277x; 1.0277x over previous
import functools

import jax
import jax.numpy as jnp
from jax import lax
from jax.experimental import pallas as pl
from jax.experimental.pallas import tpu as pltpu
from jax.experimental.pallas import tpu_sc as plsc

INF = 1e8
LEVEL_SIZES = (32768, 16384, 8192)
LEVEL_OFFS = (0, 32768, 49152)
STRIDES = (1.0, 2.0, 4.0)
B = 2
M = 256
N_TOTAL = 57344
AUDIO_TARGET_RATE = 22050.0 / 256.0
BEAT_RADIUS = 2.5
DOWNBEAT_RADIUS = 4.5
EPS = 1e-6

_EDGE0 = 0.35 + (0.7 - 0.35) / 2.0
_EDGE1 = 0.7 + (1.4 - 0.7) / 2.0
RANGE_LO = (-1.0 * AUDIO_TARGET_RATE, _EDGE0 * AUDIO_TARGET_RATE, _EDGE1 * AUDIO_TARGET_RATE)
RANGE_HI = (_EDGE0 * AUDIO_TARGET_RATE, _EDGE1 * AUDIO_TARGET_RATE, 1000.0 * AUDIO_TARGET_RATE)

NW = 32
CMAX = LEVEL_SIZES[0] // NW
_PAD = 8


def _sc_assign_body(ann_hbm, aux_hbm, out_hbm, ann_v, aux_v, ba_v, bl_v, br_v,
                    bc_v, st_v, sem):
    wid = lax.axis_index("s") * 2 + lax.axis_index("c")
    pltpu.sync_copy(ann_hbm, ann_v)
    pltpu.sync_copy(aux_hbm, aux_v)
    lane = lax.broadcasted_iota(jnp.int32, (16,), 0)
    lanef = lane.astype(jnp.float32)
    big_g = jnp.full((16,), float(M), jnp.float32)
    neg_g = jnp.full((16,), -1.0, jnp.float32)
    out_dmas = []

    for b in range(B):
        for lvl in range(3):
            C = LEVEL_SIZES[lvl] // NW
            s = STRIDES[lvl]
            inv_s = 1.0 / s
            lo = RANGE_LO[lvl]
            hi = RANGE_HI[lvl]
            base = wid * C

            def init_body(i, _):
                ba_v[pl.ds(i * 16, 16)] = jnp.full((16,), INF, jnp.float32)
                return 0

            lax.fori_loop(0, (C + 32) // 16, init_body, 0, unroll=4)

            def ann_group_body(g, _, b=b, base=base, C=C, s=s, inv_s=inv_s,
                               lo=lo, hi=hi):
                gs = pl.ds(g * 16, 16)
                lvec = ann_v[b, 0, gs]
                k0v = (lvec * inv_s).astype(jnp.int32)
                rvec = ann_v[b, 1, gs]
                cvec = ann_v[b, 2, gs]
                for j in range(16):
                    k0 = k0v[j]

                    @pl.when((k0 + 5 >= base) & (k0 < base + C))
                    def _process(j=j, k0=k0):
                        l = lvec[j]
                        r = rvec[j]
                        c = cvec[j]
                        kvec = k0 + lane
                        af = kvec.astype(jnp.float32) * s
                        zero_v = af * 0.0
                        l_v = l + zero_v
                        r_v = r + zero_v
                        c_v = c + zero_v
                        rad_s = (DOWNBEAT_RADIUS
                                 - (DOWNBEAT_RADIUS - BEAT_RADIUS) * c) * s
                        in_box = (af >= l_v) & (af <= jnp.minimum(r_v, l + rad_s + zero_v))
                        l_star = af - l_v
                        r_star = r_v - af
                        mx = jnp.maximum(l_star, r_star)
                        valid = (in_box & (mx >= lo) & (mx <= hi)
                                 & (kvec >= base) & (kvec < base + C)
                                 & (lane < 6))
                        area_v = r_v - l_v
                        off = k0 - base + _PAD
                        sl = pl.ds(off, 16)
                        ba = ba_v[sl]
                        better = valid & (area_v < ba)
                        ba_v[sl] = jnp.where(better, area_v, ba)
                        bl_v[sl] = jnp.where(better, l_v, bl_v[sl])
                        br_v[sl] = jnp.where(better, r_v, br_v[sl])
                        bc_v[sl] = jnp.where(better, c_v, bc_v[sl])

                return 0

            hf = aux_v[b, 0] * inv_s
            tf = aux_v[b, 1] * inv_s
            basef = wid * float(C)
            lim_lo = basef - 5.0
            lim_hi = basef + float(C)
            g_lo = wid * 0 + M // 16
            g_hi = wid * 0 - 1
            for g in range(M // 16 - 1, -1, -1):
                g_lo = jnp.where((tf[g] >= lim_lo) & (hf[g] < lim_hi), g, g_lo)
            for g in range(M // 16):
                g_hi = jnp.where((tf[g] >= lim_lo) & (hf[g] < lim_hi), g, g_hi)
            lax.fori_loop(g_lo, g_hi + 1, ann_group_body, 0, unroll=False)

            def epi_body(i, _, b=b, lvl=lvl, base=base, s=s, inv_s=inv_s):
                sl = pl.ds(_PAD + i * 16, 16)
                kvec = base + i * 16 + lane
                af = kvec.astype(jnp.float32) * s
                ba = ba_v[sl]
                pos = ba < INF
                posf = jnp.sign(INF - ba)
                zero_v = posf * 0.0
                bl = jnp.where(pos, bl_v[sl], zero_v)
                br = jnp.where(pos, br_v[sl], zero_v)
                bc = jnp.where(pos, bc_v[sl], zero_v)
                sbase = (b * 3 + lvl) * 3 * CMAX
                st_v[pl.ds(sbase + i * 16, 16)] = posf + bc
                st_v[pl.ds(sbase + CMAX + i * 16, 16)] = (af - bl) * inv_s
                st_v[pl.ds(sbase + 2 * CMAX + i * 16, 16)] = (br - af) * inv_s
                return 0

            lax.fori_loop(0, C // 16, epi_body, 0, unroll=2)

            gstart = LEVEL_OFFS[lvl] + base
            for f in range(3):
                lin = (b * 3 + f) * N_TOTAL + gstart
                out_dmas.append(pltpu.async_copy(
                    st_v.at[pl.ds(((b * 3 + lvl) * 3 + f) * CMAX, C)],
                    out_hbm.at[pl.ds(lin, C)], sem))

    for dma in out_dmas:
        dma.wait()


def _make_sc_assign():
    mesh = plsc.VectorSubcoreMesh(core_axis_name="c", subcore_axis_name="s")
    return pl.kernel(
        _sc_assign_body,
        out_type=jax.ShapeDtypeStruct((B * 3 * N_TOTAL,), jnp.float32),
        mesh=mesh,
        scratch_types=[
            pltpu.VMEM((B, 3, M), jnp.float32),
            pltpu.VMEM((B, 2, 16), jnp.float32),
            pltpu.VMEM((CMAX + 32,), jnp.float32),
            pltpu.VMEM((CMAX + 32,), jnp.float32),
            pltpu.VMEM((CMAX + 32,), jnp.float32),
            pltpu.VMEM((CMAX + 32,), jnp.float32),
            pltpu.VMEM((B * 3 * 3 * CMAX,), jnp.float32),
            pltpu.SemaphoreType.DMA,
        ],
    )


BLK = 14336
NBLK = N_TOTAL // BLK


def _loss_kernel(cls_ref, reg_ref, lef_ref, asg_ref, out_ref, acc_ref):
    j = pl.program_id(1)

    pc = asg_ref[0, 0]
    nl = asg_ref[0, 1]
    nr = asg_ref[0, 2]

    posf = jnp.minimum(pc, 1.0)
    t1 = jnp.maximum(pc - 1.0, 0.0)
    t0 = posf - t1
    p = jnp.clip(cls_ref[0], EPS, 1.0 - EPS)
    p0 = p[0]
    p1 = p[1]

    def _focal(t, q):
        is_pos = t == 1.0
        arg = jnp.where(is_pos, q, 1.0 - q)
        coef = jnp.where(is_pos, 0.25 * (1.0 - q) * (1.0 - q), 0.75 * q * q)
        return coef * (-jnp.log(arg))

    cls_sum = jnp.sum(_focal(t0, p0) + _focal(t1, p1))

    tl = jnp.maximum(nl, 1e-3)
    tr = jnp.maximum(nr, 1e-3)
    reg = reg_ref[0]
    pl_ = reg[0]
    pr_ = reg[1]
    inter = jnp.minimum(pl_, tl) + jnp.minimum(pr_, tr)
    union = jnp.maximum(pl_, tl) + jnp.maximum(pr_, tr)
    iou = jnp.clip(inter / (union + EPS), EPS, 1.0)
    reg_sum = jnp.sum(-jnp.log(iou) * posf)

    lt = jnp.clip(tr / (tl + tr + EPS), EPS, 1.0 - EPS)
    lp = jnp.clip(lef_ref[0, 0], EPS, 1.0 - EPS)
    lbce = -(lt * jnp.log(lp) + (1.0 - lt) * jnp.log(1.0 - lp))
    left_sum = jnp.sum(lbce * posf)

    pos_sum = jnp.sum(posf)

    b = pl.program_id(0)
    lane = lax.broadcasted_iota(jnp.int32, (1, 1, 128), 2)
    acc = jnp.where(lane == 0, cls_sum + reg_sum + left_sum,
                    jnp.where(lane == 1, pos_sum, 0.0))

    @pl.when(j == 0)
    def _init():
        acc_ref[pl.ds(b, 1)] = acc

    @pl.when(j > 0)
    def _acc():
        acc_ref[pl.ds(b, 1)] += acc

    @pl.when((b == B - 1) & (j == NBLK - 1))
    def _finalize():
        total = 0.0
        for bb in range(B):
            row = acc_ref[bb]
            lane2 = lax.broadcasted_iota(jnp.int32, (1, 128), 1)
            lsum = jnp.sum(jnp.where(lane2 == 0, row, 0.0))
            npos = jnp.maximum(jnp.sum(jnp.where(lane2 == 1, row, 0.0)), 1.0)
            total = total + lsum / npos
        out_ref[...] = jnp.where(lane[0] == 0, total / float(B), 0.0)


def kernel(classifications, regressions, leftnesses, annotations):
    cls_t = classifications.transpose(0, 2, 1)
    reg_t = regressions.transpose(0, 2, 1)
    lef_t = leftnesses.transpose(0, 2, 1)
    ann_t = annotations.transpose(0, 2, 1)
    starts_t = ann_t[:, 0, :]
    aux = jnp.stack([starts_t[:, 0::16], starts_t[:, 15::16]], axis=1)

    assign = _make_sc_assign()(ann_t, aux).reshape(B, 3, N_TOTAL)

    out = pl.pallas_call(
        _loss_kernel,
        grid=(B, NBLK),
        in_specs=[
            pl.BlockSpec((1, 2, BLK), lambda b, j: (b, 0, j)),
            pl.BlockSpec((1, 2, BLK), lambda b, j: (b, 0, j)),
            pl.BlockSpec((1, 1, BLK), lambda b, j: (b, 0, j)),
            pl.BlockSpec((1, 3, BLK), lambda b, j: (b, 0, j)),
        ],
        out_specs=pl.BlockSpec((1, 128), lambda b, j: (0, 0)),
        out_shape=jax.ShapeDtypeStruct((1, 128), jnp.float32),
        scratch_shapes=[pltpu.VMEM((B, 1, 128), jnp.float32)],
    )(cls_t, reg_t, lef_t, assign)

    return out[0, 0]


# speedup vs baseline: 1.1028x; 1.0730x over previous
import functools

import jax
import jax.numpy as jnp
from jax import lax
from jax.experimental import pallas as pl
from jax.experimental.pallas import tpu as pltpu
from jax.experimental.pallas import tpu_sc as plsc

INF = 1e8
LEVEL_SIZES = (32768, 16384, 8192)
LEVEL_OFFS = (0, 32768, 49152)
STRIDES = (1.0, 2.0, 4.0)
B = 2
M = 256
N_TOTAL = 57344
AUDIO_TARGET_RATE = 22050.0 / 256.0
BEAT_RADIUS = 2.5
DOWNBEAT_RADIUS = 4.5
EPS = 1e-6

_EDGE0 = 0.35 + (0.7 - 0.35) / 2.0
_EDGE1 = 0.7 + (1.4 - 0.7) / 2.0
RANGE_LO = (-1.0 * AUDIO_TARGET_RATE, _EDGE0 * AUDIO_TARGET_RATE, _EDGE1 * AUDIO_TARGET_RATE)
RANGE_HI = (_EDGE0 * AUDIO_TARGET_RATE, _EDGE1 * AUDIO_TARGET_RATE, 1000.0 * AUDIO_TARGET_RATE)

NW = 32
CMAX = LEVEL_SIZES[0] // NW
_PAD = 8


def _sc_assign_body(ann_hbm, aux_hbm, out_hbm, ann_v, aux_v, ba_v, bl_v, br_v,
                    bc_v, st_v, sem):
    wid = lax.axis_index("s") * 2 + lax.axis_index("c")
    pltpu.sync_copy(ann_hbm, ann_v)
    pltpu.sync_copy(aux_hbm, aux_v)
    lane = lax.broadcasted_iota(jnp.int32, (16,), 0)
    lanef = lane.astype(jnp.float32)
    big_g = jnp.full((16,), float(M), jnp.float32)
    neg_g = jnp.full((16,), -1.0, jnp.float32)
    out_dmas = []

    for b in range(B):
        for lvl in range(3):
            C = LEVEL_SIZES[lvl] // NW
            s = STRIDES[lvl]
            inv_s = 1.0 / s
            lo = RANGE_LO[lvl]
            hi = RANGE_HI[lvl]
            base = wid * C

            def init_body(i, _):
                ba_v[pl.ds(i * 16, 16)] = jnp.full((16,), INF, jnp.float32)
                return 0

            lax.fori_loop(0, (C + 32) // 16, init_body, 0, unroll=4)

            def ann_group_body(g, _, b=b, base=base, C=C, s=s, inv_s=inv_s,
                               lo=lo, hi=hi):
                gs = pl.ds(g * 16, 16)
                lvec = ann_v[b, 0, gs]
                k0v = (lvec * inv_s).astype(jnp.int32)
                rvec = ann_v[b, 1, gs]
                cvec = ann_v[b, 2, gs]
                for j in range(16):
                    k0 = k0v[j]

                    @pl.when((k0 + 5 >= base) & (k0 < base + C))
                    def _process(j=j, k0=k0):
                        l = lvec[j]
                        r = rvec[j]
                        c = cvec[j]
                        kvec = k0 + lane
                        af = kvec.astype(jnp.float32) * s
                        zero_v = af * 0.0
                        l_v = l + zero_v
                        r_v = r + zero_v
                        c_v = c + zero_v
                        rad_s = (DOWNBEAT_RADIUS
                                 - (DOWNBEAT_RADIUS - BEAT_RADIUS) * c) * s
                        in_box = (af >= l_v) & (af <= jnp.minimum(r_v, l + rad_s + zero_v))
                        l_star = af - l_v
                        r_star = r_v - af
                        mx = jnp.maximum(l_star, r_star)
                        valid = (in_box & (mx >= lo) & (mx <= hi)
                                 & (kvec >= base) & (kvec < base + C)
                                 & (lane < 6))
                        area_v = r_v - l_v
                        off = k0 - base + _PAD
                        sl = pl.ds(off, 16)
                        ba = ba_v[sl]
                        better = valid & (area_v < ba)
                        ba_v[sl] = jnp.where(better, area_v, ba)
                        bl_v[sl] = jnp.where(better, l_v, bl_v[sl])
                        br_v[sl] = jnp.where(better, r_v, br_v[sl])
                        bc_v[sl] = jnp.where(better, c_v, bc_v[sl])

                return 0

            hf = aux_v[b, 0] * inv_s
            tf = aux_v[b, 1] * inv_s
            basef = wid * float(C)
            lim_lo = basef - 5.0
            lim_hi = basef + float(C)
            g_lo = wid * 0 + M // 16
            g_hi = wid * 0 - 1
            for g in range(M // 16 - 1, -1, -1):
                g_lo = jnp.where((tf[g] >= lim_lo) & (hf[g] < lim_hi), g, g_lo)
            for g in range(M // 16):
                g_hi = jnp.where((tf[g] >= lim_lo) & (hf[g] < lim_hi), g, g_hi)
            lax.fori_loop(g_lo, g_hi + 1, ann_group_body, 0, unroll=False)

            def epi_body(i, _, b=b, lvl=lvl, base=base, s=s, inv_s=inv_s):
                sl = pl.ds(_PAD + i * 16, 16)
                kvec = base + i * 16 + lane
                af = kvec.astype(jnp.float32) * s
                ba = ba_v[sl]
                pos = ba < INF
                posf = jnp.sign(INF - ba)
                zero_v = posf * 0.0
                bl = jnp.where(pos, bl_v[sl], zero_v)
                br = jnp.where(pos, br_v[sl], zero_v)
                bc = jnp.where(pos, bc_v[sl], zero_v)
                sbase = (b * 3 + lvl) * 3 * CMAX
                st_v[pl.ds(sbase + i * 16, 16)] = posf + bc
                st_v[pl.ds(sbase + CMAX + i * 16, 16)] = (af - bl) * inv_s
                st_v[pl.ds(sbase + 2 * CMAX + i * 16, 16)] = (br - af) * inv_s
                return 0

            lax.fori_loop(0, C // 16, epi_body, 0, unroll=2)

            gstart = LEVEL_OFFS[lvl] + base
            for f in range(3):
                lin = (b * 3 + f) * N_TOTAL + gstart
                out_dmas.append(pltpu.async_copy(
                    st_v.at[pl.ds(((b * 3 + lvl) * 3 + f) * CMAX, C)],
                    out_hbm.at[pl.ds(lin, C)], sem))

    for dma in out_dmas:
        dma.wait()


def _make_sc_assign():
    mesh = plsc.VectorSubcoreMesh(core_axis_name="c", subcore_axis_name="s")
    return pl.kernel(
        _sc_assign_body,
        out_type=jax.ShapeDtypeStruct((B * 3 * N_TOTAL,), jnp.float32),
        mesh=mesh,
        scratch_types=[
            pltpu.VMEM((B, 3, M), jnp.float32),
            pltpu.VMEM((B, 2, 16), jnp.float32),
            pltpu.VMEM((CMAX + 32,), jnp.float32),
            pltpu.VMEM((CMAX + 32,), jnp.float32),
            pltpu.VMEM((CMAX + 32,), jnp.float32),
            pltpu.VMEM((CMAX + 32,), jnp.float32),
            pltpu.VMEM((B * 3 * 3 * CMAX,), jnp.float32),
            pltpu.SemaphoreType.DMA,
        ],
    )


BLK = 14336
NBLK = N_TOTAL // BLK


def _loss_kernel(p0_ref, p1_ref, pl_ref, pr_ref, lef_ref, pc_ref, nl_ref,
                 nr_ref, out_ref, acc_ref):
    j = pl.program_id(1)

    pc = pc_ref[...]
    nl = nl_ref[...]
    nr = nr_ref[...]

    posf = jnp.minimum(pc, 1.0)
    t1 = jnp.maximum(pc - 1.0, 0.0)
    t0 = posf - t1
    p0 = jnp.clip(p0_ref[...], EPS, 1.0 - EPS)
    p1 = jnp.clip(p1_ref[...], EPS, 1.0 - EPS)

    def _focal(t, q):
        is_pos = t == 1.0
        arg = jnp.where(is_pos, q, 1.0 - q)
        coef = jnp.where(is_pos, 0.25 * (1.0 - q) * (1.0 - q), 0.75 * q * q)
        return coef * (-jnp.log(arg))

    cls_sum = jnp.sum(_focal(t0, p0) + _focal(t1, p1))

    tl = jnp.maximum(nl, 1e-3)
    tr = jnp.maximum(nr, 1e-3)
    pl_ = pl_ref[...]
    pr_ = pr_ref[...]
    inter = jnp.minimum(pl_, tl) + jnp.minimum(pr_, tr)
    union = jnp.maximum(pl_, tl) + jnp.maximum(pr_, tr)
    iou = jnp.clip(inter / (union + EPS), EPS, 1.0)
    reg_sum = jnp.sum(-jnp.log(iou) * posf)

    lt = jnp.clip(tr / (tl + tr + EPS), EPS, 1.0 - EPS)
    lp = jnp.clip(lef_ref[...], EPS, 1.0 - EPS)
    lbce = -(lt * jnp.log(lp) + (1.0 - lt) * jnp.log(1.0 - lp))
    left_sum = jnp.sum(lbce * posf)

    pos_sum = jnp.sum(posf)

    b = pl.program_id(0)
    lane = lax.broadcasted_iota(jnp.int32, (1, 1, 128), 2)
    acc = jnp.where(lane == 0, cls_sum + reg_sum + left_sum,
                    jnp.where(lane == 1, pos_sum, 0.0))

    @pl.when(j == 0)
    def _init():
        acc_ref[pl.ds(b, 1)] = acc

    @pl.when(j > 0)
    def _acc():
        acc_ref[pl.ds(b, 1)] += acc

    @pl.when((b == B - 1) & (j == NBLK - 1))
    def _finalize():
        total = 0.0
        for bb in range(B):
            row = acc_ref[bb]
            lane2 = lax.broadcasted_iota(jnp.int32, (1, 128), 1)
            lsum = jnp.sum(jnp.where(lane2 == 0, row, 0.0))
            npos = jnp.maximum(jnp.sum(jnp.where(lane2 == 1, row, 0.0)), 1.0)
            total = total + lsum / npos
        out_ref[...] = jnp.where(lane[0] == 0, total / float(B), 0.0)


def kernel(classifications, regressions, leftnesses, annotations):
    cls_f = classifications.transpose(0, 2, 1).reshape(-1)
    reg_f = regressions.transpose(0, 2, 1).reshape(-1)
    lef_f = leftnesses.reshape(-1)
    ann_t = annotations.transpose(0, 2, 1)
    starts_t = ann_t[:, 0, :]
    aux = jnp.stack([starts_t[:, 0::16], starts_t[:, 15::16]], axis=1)

    assign = _make_sc_assign()(ann_t, aux)

    def _fspec(nf, f):
        return pl.BlockSpec((BLK,), lambda b, j, nf=nf, f=f: ((b * nf + f) * NBLK + j,))

    out = pl.pallas_call(
        _loss_kernel,
        grid=(B, NBLK),
        in_specs=[
            _fspec(2, 0), _fspec(2, 1),
            _fspec(2, 0), _fspec(2, 1),
            _fspec(1, 0),
            _fspec(3, 0), _fspec(3, 1), _fspec(3, 2),
        ],
        out_specs=pl.BlockSpec((1, 128), lambda b, j: (0, 0)),
        out_shape=jax.ShapeDtypeStruct((1, 128), jnp.float32),
        scratch_shapes=[pltpu.VMEM((B, 1, 128), jnp.float32)],
    )(cls_f, cls_f, reg_f, reg_f, lef_f, assign, assign, assign)

    return out[0, 0]


# speedup vs baseline: 1.2227x; 1.1088x over previous
import functools

import jax
import jax.numpy as jnp
from jax import lax
from jax.experimental import pallas as pl
from jax.experimental.pallas import tpu as pltpu
from jax.experimental.pallas import tpu_sc as plsc

INF = 1e8
LEVEL_SIZES = (32768, 16384, 8192)
LEVEL_OFFS = (0, 32768, 49152)
STRIDES = (1.0, 2.0, 4.0)
B = 2
M = 256
N_TOTAL = 57344
AUDIO_TARGET_RATE = 22050.0 / 256.0
BEAT_RADIUS = 2.5
DOWNBEAT_RADIUS = 4.5
EPS = 1e-6

_EDGE0 = 0.35 + (0.7 - 0.35) / 2.0
_EDGE1 = 0.7 + (1.4 - 0.7) / 2.0
RANGE_LO = (-1.0 * AUDIO_TARGET_RATE, _EDGE0 * AUDIO_TARGET_RATE, _EDGE1 * AUDIO_TARGET_RATE)
RANGE_HI = (_EDGE0 * AUDIO_TARGET_RATE, _EDGE1 * AUDIO_TARGET_RATE, 1000.0 * AUDIO_TARGET_RATE)

NW = 32
CMAX = LEVEL_SIZES[0] // NW
_PAD = 8


def _sc_assign_body(ann_hbm, out_hbm, ann_v, ba_v, bl_v, br_v, bc_v, st_v, sem):
    wid = lax.axis_index("s") * 2 + lax.axis_index("c")
    pltpu.sync_copy(ann_hbm, ann_v)
    lane = lax.broadcasted_iota(jnp.int32, (16,), 0)

    def batch_body(b, carry):
        heads, tails = [], []
        for g in range(M // 16):
            lv = ann_v[b, 0, pl.ds(g * 16, 16)]
            heads.append(lv[0])
            tails.append(lv[15])

        for lvl in range(3):
            C = LEVEL_SIZES[lvl] // NW
            s = STRIDES[lvl]
            inv_s = 1.0 / s
            lo = RANGE_LO[lvl]
            hi = RANGE_HI[lvl]
            base = wid * C

            def init_body(i, _):
                ba_v[pl.ds(i * 16, 16)] = jnp.full((16,), INF, jnp.float32)
                return 0

            lax.fori_loop(0, (C + 32) // 16, init_body, 0, unroll=4)

            def ann_group_body(g, _, b=b, base=base, C=C, s=s, inv_s=inv_s,
                               lo=lo, hi=hi):
                gs = pl.ds(g * 16, 16)
                lvec = ann_v[b, 0, gs]
                k0v = (lvec * inv_s).astype(jnp.int32)
                rvec = ann_v[b, 1, gs]
                cvec = ann_v[b, 2, gs]
                for j in range(16):
                    k0 = k0v[j]

                    @pl.when((k0 + 5 >= base) & (k0 < base + C))
                    def _process(j=j, k0=k0):
                        l = lvec[j]
                        r = rvec[j]
                        c = cvec[j]
                        kvec = k0 + lane
                        af = kvec.astype(jnp.float32) * s
                        zero_v = af * 0.0
                        l_v = l + zero_v
                        r_v = r + zero_v
                        c_v = c + zero_v
                        rad_s = (DOWNBEAT_RADIUS
                                 - (DOWNBEAT_RADIUS - BEAT_RADIUS) * c) * s
                        in_box = (af >= l_v) & (af <= jnp.minimum(r_v, l + rad_s + zero_v))
                        l_star = af - l_v
                        r_star = r_v - af
                        mx = jnp.maximum(l_star, r_star)
                        valid = (in_box & (mx >= lo) & (mx <= hi)
                                 & (kvec >= base) & (kvec < base + C)
                                 & (lane < 6))
                        area_v = r_v - l_v
                        off = k0 - base + _PAD
                        sl = pl.ds(off, 16)
                        ba = ba_v[sl]
                        better = valid & (area_v < ba)
                        ba_v[sl] = jnp.where(better, area_v, ba)
                        bl_v[sl] = jnp.where(better, l_v, bl_v[sl])
                        br_v[sl] = jnp.where(better, r_v, br_v[sl])
                        bc_v[sl] = jnp.where(better, c_v, bc_v[sl])

                return 0

            lim_lo = (wid * C - 5).astype(jnp.float32) * s
            lim_hi = (wid * C + C).astype(jnp.float32) * s
            g_lo = wid * 0 + M // 16
            g_hi = wid * 0 - 1
            for g in range(M // 16 - 1, -1, -1):
                g_lo = jnp.where((tails[g] >= lim_lo) & (heads[g] < lim_hi),
                                 g, g_lo)
            for g in range(M // 16):
                g_hi = jnp.where((tails[g] >= lim_lo) & (heads[g] < lim_hi),
                                 g, g_hi)
            lax.fori_loop(g_lo, g_hi + 1, ann_group_body, 0, unroll=False)

            def epi_body(i, _, b=b, lvl=lvl, base=base, s=s, inv_s=inv_s):
                sl = pl.ds(_PAD + i * 16, 16)
                kvec = base + i * 16 + lane
                af = kvec.astype(jnp.float32) * s
                ba = ba_v[sl]
                pos = ba < INF
                posf = jnp.sign(INF - ba)
                zero_v = posf * 0.0
                bl = jnp.where(pos, bl_v[sl], zero_v)
                br = jnp.where(pos, br_v[sl], zero_v)
                bc = jnp.where(pos, bc_v[sl], zero_v)
                sbase = (b * 3 + lvl) * 3 * CMAX
                st_v[pl.ds(sbase + i * 16, 16)] = posf + bc
                st_v[pl.ds(sbase + CMAX + i * 16, 16)] = (af - bl) * inv_s
                st_v[pl.ds(sbase + 2 * CMAX + i * 16, 16)] = (br - af) * inv_s
                return 0

            lax.fori_loop(0, C // 16, epi_body, 0, unroll=2)

            gstart = LEVEL_OFFS[lvl] + base
            for f in range(3):
                lin = (b * 3 + f) * N_TOTAL + gstart
                pltpu.async_copy(
                    st_v.at[pl.ds((b * 9 + lvl * 3 + f) * CMAX, C)],
                    out_hbm.at[pl.ds(lin, C)], sem)
        return carry

    lax.fori_loop(0, B, batch_body, 0, unroll=False)

    for _b in range(B):
        for _lvl in range(3):
            _C = LEVEL_SIZES[_lvl] // NW
            for _f in range(3):
                pltpu.make_async_copy(
                    st_v.at[pl.ds(0, _C)],
                    out_hbm.at[pl.ds(wid * _C, _C)], sem).wait()


def _make_sc_assign():
    mesh = plsc.VectorSubcoreMesh(core_axis_name="c", subcore_axis_name="s")
    return pl.kernel(
        _sc_assign_body,
        out_type=jax.ShapeDtypeStruct((B * 3 * N_TOTAL,), jnp.float32),
        mesh=mesh,
        scratch_types=[
            pltpu.VMEM((B, 3, M), jnp.float32),
            pltpu.VMEM((CMAX + 32,), jnp.float32),
            pltpu.VMEM((CMAX + 32,), jnp.float32),
            pltpu.VMEM((CMAX + 32,), jnp.float32),
            pltpu.VMEM((CMAX + 32,), jnp.float32),
            pltpu.VMEM((B * 3 * 3 * CMAX,), jnp.float32),
            pltpu.SemaphoreType.DMA,
        ],
    )


BLK = 14336
NBLK = N_TOTAL // BLK


def _loss_kernel(p0_ref, p1_ref, pl_ref, pr_ref, lef_ref, pc_ref, nl_ref,
                 nr_ref, out_ref, acc_ref):
    j = pl.program_id(1)

    pc = pc_ref[...]
    nl = nl_ref[...]
    nr = nr_ref[...]

    posf = jnp.minimum(pc, 1.0)
    t1 = jnp.maximum(pc - 1.0, 0.0)
    t0 = posf - t1
    p0 = jnp.clip(p0_ref[...], EPS, 1.0 - EPS)
    p1 = jnp.clip(p1_ref[...], EPS, 1.0 - EPS)

    def _focal(t, q):
        is_pos = t == 1.0
        arg = jnp.where(is_pos, q, 1.0 - q)
        coef = jnp.where(is_pos, 0.25 * (1.0 - q) * (1.0 - q), 0.75 * q * q)
        return coef * (-jnp.log(arg))

    cls_sum = jnp.sum(_focal(t0, p0) + _focal(t1, p1))

    tl = jnp.maximum(nl, 1e-3)
    tr = jnp.maximum(nr, 1e-3)
    pl_ = pl_ref[...]
    pr_ = pr_ref[...]
    inter = jnp.minimum(pl_, tl) + jnp.minimum(pr_, tr)
    union = jnp.maximum(pl_, tl) + jnp.maximum(pr_, tr)
    iou = jnp.clip(inter / (union + EPS), EPS, 1.0)
    reg_sum = jnp.sum(-jnp.log(iou) * posf)

    lt = jnp.clip(tr / (tl + tr + EPS), EPS, 1.0 - EPS)
    lp = jnp.clip(lef_ref[...], EPS, 1.0 - EPS)
    lbce = -(lt * jnp.log(lp) + (1.0 - lt) * jnp.log(1.0 - lp))
    left_sum = jnp.sum(lbce * posf)

    pos_sum = jnp.sum(posf)

    b = pl.program_id(0)
    lane = lax.broadcasted_iota(jnp.int32, (1, 1, 128), 2)
    acc = jnp.where(lane == 0, cls_sum + reg_sum + left_sum,
                    jnp.where(lane == 1, pos_sum, 0.0))

    @pl.when(j == 0)
    def _init():
        acc_ref[pl.ds(b, 1)] = acc

    @pl.when(j > 0)
    def _acc():
        acc_ref[pl.ds(b, 1)] += acc

    @pl.when((b == B - 1) & (j == NBLK - 1))
    def _finalize():
        total = 0.0
        for bb in range(B):
            row = acc_ref[bb]
            lane2 = lax.broadcasted_iota(jnp.int32, (1, 128), 1)
            lsum = jnp.sum(jnp.where(lane2 == 0, row, 0.0))
            npos = jnp.maximum(jnp.sum(jnp.where(lane2 == 1, row, 0.0)), 1.0)
            total = total + lsum / npos
        out_ref[...] = jnp.where(lane[0] == 0, total / float(B), 0.0)


def kernel(classifications, regressions, leftnesses, annotations):
    cls_f = classifications.transpose(0, 2, 1).reshape(-1)
    reg_f = regressions.transpose(0, 2, 1).reshape(-1)
    lef_f = leftnesses.reshape(-1)
    ann_t = annotations.transpose(0, 2, 1)

    assign = _make_sc_assign()(ann_t)

    def _fspec(nf, f):
        return pl.BlockSpec((BLK,), lambda b, j, nf=nf, f=f: ((b * nf + f) * NBLK + j,))

    out = pl.pallas_call(
        _loss_kernel,
        grid=(B, NBLK),
        in_specs=[
            _fspec(2, 0), _fspec(2, 1),
            _fspec(2, 0), _fspec(2, 1),
            _fspec(1, 0),
            _fspec(3, 0), _fspec(3, 1), _fspec(3, 2),
        ],
        out_specs=pl.BlockSpec((1, 128), lambda b, j: (0, 0)),
        out_shape=jax.ShapeDtypeStruct((1, 128), jnp.float32),
        scratch_shapes=[pltpu.VMEM((B, 1, 128), jnp.float32)],
    )(cls_f, cls_f, reg_f, reg_f, lef_f, assign, assign, assign)

    return out[0, 0]
